# baseline (speedup 1.0000x reference)
import functools

import jax
import jax.numpy as jnp
from jax import lax
from jax.experimental import pallas as pl
from jax.experimental.pallas import tpu as pltpu
from jax.experimental.pallas import tpu_sc as plsc

B = 16384
CTX = 128
D = 128
H = 128
K = 8
RH = 64
ROWS_A = 4096
ROWS_C = 2048
NC = 2
NS = 16
L = 16
NW = NC * NS
STRIP = B // NW


def _router_kernel(ctx_ref, gt_ref, rw1_ref, rb1_ref, lng_ref, lnb_ref,
                   rw2_ref, rb2_ref, rw3_ref, rb3_ref, prior_ref,
                   probs_ref, sel_ref):
    f32 = jnp.float32
    dn_lane_lane = (((1,), (1,)), ((), ()))
    dn_lane_sub = (((1,), (0,)), ((), ()))
    ht = jax.lax.dot_general(rw1_ref[...], ctx_ref[...], dn_lane_lane,
                             preferred_element_type=f32) + rb1_ref[...]
    mu = jnp.mean(ht, axis=0, keepdims=True)
    var = jnp.mean((ht - mu) ** 2, axis=0, keepdims=True)
    ht = (ht - mu) / jnp.sqrt(var + 1e-5) * lng_ref[...] + lnb_ref[...]
    ht = jax.nn.relu(ht)
    h2t = jax.nn.relu(
        jax.lax.dot_general(rw2_ref[...], ht, dn_lane_sub,
                            preferred_element_type=f32) + rb2_ref[...])
    logits = jax.lax.dot_general(rw3_ref[...], h2t, dn_lane_sub,
                                 preferred_element_type=f32) + rb3_ref[...]
    logits = logits + prior_ref[...]
    probs_ref[...] = jax.nn.sigmoid(logits)
    sel_ref[...] = jax.nn.sigmoid(logits + gt_ref[...])


def _sc_gate_kernel(probs_hbm, sel_hbm, w_hbm, selout_hbm, coef_hbm,
                    pb, sb, so, cf, wv):
    f32 = jnp.float32
    i32 = jnp.int32
    wid = lax.axis_index("s") * NC + lax.axis_index("c")
    base = wid * STRIP
    pltpu.sync_copy(w_hbm.at[...], wv.at[...])
    for k in range(K):
        pltpu.sync_copy(probs_hbm.at[pl.ds(k * B + base, STRIP)],
                        pb.at[pl.ds(k * STRIP, STRIP)])
        pltpu.sync_copy(sel_hbm.at[pl.ds(k * B + base, STRIP)],
                        sb.at[pl.ds(k * STRIP, STRIP)])
    for j in range(STRIP // L):
        off = j * L
        p = [pb[pl.ds(k * STRIP + off, L)] for k in range(K)]
        m1 = p[0]
        i1 = jnp.zeros((L,), i32)
        m2 = jnp.full((L,), -jnp.inf, f32)
        i2 = jnp.full((L,), K, i32)
        for k in range(1, K):
            kv = jnp.full((L,), k, i32)
            gt1 = p[k] > m1
            gt2 = p[k] > m2
            m2 = jnp.where(gt1, m1, jnp.where(gt2, p[k], m2))
            i2 = jnp.where(gt1, i1, jnp.where(gt2, kv, i2))
            m1 = jnp.where(gt1, p[k], m1)
            i1 = jnp.where(gt1, kv, i1)
        one = jnp.ones((L,), f32)
        zero = jnp.zeros((L,), f32)
        for k in range(K):
            kv = jnp.full((L,), k, i32)
            s = sb[pl.ds(k * STRIP + off, L)]
            mask = (i1 == kv) | (i2 == kv)
            sel_out = jnp.where(mask, one, s)
            c = jnp.where(sel_out > 0.5, sel_out, zero) * wv[pl.ds(k * L, L)]
            so[pl.ds(k * STRIP + off, L)] = sel_out
            cf[pl.ds(k * STRIP + off, L)] = c
    for k in range(K):
        pltpu.sync_copy(so.at[pl.ds(k * STRIP, STRIP)],
                        selout_hbm.at[pl.ds(k * B + base, STRIP)])
        pltpu.sync_copy(cf.at[pl.ds(k * STRIP, STRIP)],
                        coef_hbm.at[pl.ds(k * B + base, STRIP)])


def _encode_kernel(m0, m1, m2, m3, m4, m5, m6, m7, coeft_ref, selt_ref,
                   probst_ref, encw_ref, encb_ref,
                   fused_ref, sel_ref, probs_ref):
    f32 = jnp.float32
    dn_lane_lane = (((1,), (1,)), ((), ()))
    coef_t = coeft_ref[...]
    mods = (m0, m1, m2, m3, m4, m5, m6, m7)
    acc_t = None
    for k in range(K):
        enc_t = jax.lax.dot_general(encw_ref[k].astype(jnp.bfloat16),
                                    mods[k][...].astype(jnp.bfloat16),
                                    dn_lane_lane,
                                    preferred_element_type=f32)
        term = coef_t[k:k + 1, :] * enc_t
        acc_t = term if acc_t is None else acc_t + term
    bias = jnp.dot(coef_t.T, encb_ref[...], preferred_element_type=f32)
    fused_ref[...] = acc_t.T + bias
    sel_ref[...] = selt_ref[...].T
    probs_ref[...] = probst_ref[...].T


@jax.jit
def kernel(context, mod_0, mod_1, mod_2, mod_3, mod_4, mod_5, mod_6, mod_7,
           r_w1, r_b1, ln_g, ln_b, r_w2, r_b2, r_w3, r_b3, prior, enc_W,
           enc_b, fusion_w):
    f32 = jnp.float32
    u = jax.random.uniform(jax.random.key(1234), (B, K), dtype=f32)
    g_t = (-jnp.log(-jnp.log(u + 1e-8) + 1e-8)).T

    row = lambda shape: pl.BlockSpec(shape, lambda i: (i, 0))
    colblk = lambda shape: pl.BlockSpec(shape, lambda i: (0, i))
    full2 = lambda shape: pl.BlockSpec(shape, lambda i: (0, 0))

    probs_t, selsoft_t = pl.pallas_call(
        _router_kernel,
        grid=(B // ROWS_A,),
        in_specs=[row((ROWS_A, CTX)), colblk((K, ROWS_A)),
                  full2((RH, CTX)), full2((RH, 1)), full2((RH, 1)),
                  full2((RH, 1)), full2((RH // 2, RH)), full2((RH // 2, 1)),
                  full2((K, RH // 2)), full2((K, 1)), full2((K, 1))],
        out_specs=(colblk((K, ROWS_A)), colblk((K, ROWS_A))),
        out_shape=(jax.ShapeDtypeStruct((K, B), f32),
                   jax.ShapeDtypeStruct((K, B), f32)),
        compiler_params=pltpu.CompilerParams(
            dimension_semantics=("parallel",)),
    )(context, g_t, r_w1, r_b1.reshape(RH, 1), ln_g.reshape(RH, 1),
      ln_b.reshape(RH, 1), r_w2, r_b2.reshape(RH // 2, 1), r_w3,
      r_b3.reshape(K, 1), prior.reshape(K, 1))

    w_flat = jnp.broadcast_to(
        jax.nn.softmax(fusion_w)[:, None], (K, L)).reshape(-1)
    mesh = plsc.VectorSubcoreMesh(core_axis_name="c", subcore_axis_name="s")
    sel_flat, coef_flat = pl.kernel(
        _sc_gate_kernel,
        out_type=(jax.ShapeDtypeStruct((K * B,), f32),
                  jax.ShapeDtypeStruct((K * B,), f32)),
        mesh=mesh,
        scratch_types=[
            pltpu.VMEM((K * STRIP,), f32),
            pltpu.VMEM((K * STRIP,), f32),
            pltpu.VMEM((K * STRIP,), f32),
            pltpu.VMEM((K * STRIP,), f32),
            pltpu.VMEM((K * L,), f32),
        ],
    )(probs_t.reshape(-1), selsoft_t.reshape(-1), w_flat)
    sel_t = sel_flat.reshape(K, B)
    coef_t = coef_flat.reshape(K, B)

    fused, sel, probs = pl.pallas_call(
        _encode_kernel,
        grid=(B // ROWS_C,),
        in_specs=[row((ROWS_C, D))] * K +
                 [colblk((K, ROWS_C)), colblk((K, ROWS_C)),
                  colblk((K, ROWS_C)),
                  pl.BlockSpec((K, H, D), lambda i: (0, 0, 0)),
                  full2((K, H))],
        out_specs=(row((ROWS_C, H)), row((ROWS_C, K)), row((ROWS_C, K))),
        out_shape=(jax.ShapeDtypeStruct((B, H), f32),
                   jax.ShapeDtypeStruct((B, K), f32),
                   jax.ShapeDtypeStruct((B, K), f32)),
        compiler_params=pltpu.CompilerParams(
            dimension_semantics=("parallel",)),
    )(mod_0, mod_1, mod_2, mod_3, mod_4, mod_5, mod_6, mod_7,
      coef_t, sel_t, probs_t, enc_W, enc_b)
    return fused, sel, probs


# speedup vs baseline: 1.7482x; 1.7482x over previous
import jax
import jax.numpy as jnp
from jax.experimental import pallas as pl
from jax.experimental.pallas import tpu as pltpu

B = 16384
CTX = 128
D = 128
H = 128
K = 8
RH = 64
ROWS = 2048


def _fused_kernel(ctx_ref, m0, m1, m2, m3, m4, m5, m6, m7, gt_ref,
                  rw1_ref, rb1_ref, lng_ref, lnb_ref, rw2_ref, rb2_ref,
                  rw3_ref, rb3_ref, prior_ref, encw_ref, encb_ref, fw_ref,
                  fused_ref, sel_ref, probs_ref):
    f32 = jnp.float32
    dn_lane_lane = (((1,), (1,)), ((), ()))
    dn_lane_sub = (((1,), (0,)), ((), ()))

    ht = jax.lax.dot_general(rw1_ref[...], ctx_ref[...], dn_lane_lane,
                             preferred_element_type=f32) + rb1_ref[...]
    mu = jnp.mean(ht, axis=0, keepdims=True)
    var = jnp.mean((ht - mu) ** 2, axis=0, keepdims=True)
    ht = (ht - mu) / jnp.sqrt(var + 1e-5) * lng_ref[...] + lnb_ref[...]
    ht = jax.nn.relu(ht)
    h2t = jax.nn.relu(
        jax.lax.dot_general(rw2_ref[...], ht, dn_lane_sub,
                            preferred_element_type=f32) + rb2_ref[...])
    logits = jax.lax.dot_general(rw3_ref[...], h2t, dn_lane_sub,
                                 preferred_element_type=f32) + rb3_ref[...]
    logits = logits + prior_ref[...]
    probs_t = jax.nn.sigmoid(logits)
    sel_t = jax.nn.sigmoid(logits + gt_ref[...])

    iota = jax.lax.broadcasted_iota(jnp.int32, (K, ROWS), 0)
    m1v = jnp.max(probs_t, axis=0, keepdims=True)
    i1 = jnp.min(jnp.where(probs_t == m1v, iota, K), axis=0, keepdims=True)
    p2 = jnp.where(iota == i1, -jnp.inf, probs_t)
    m2v = jnp.max(p2, axis=0, keepdims=True)
    i2 = jnp.min(jnp.where(p2 == m2v, iota, K), axis=0, keepdims=True)
    minmask = (iota == i1) | (iota == i2)
    sel_t = jnp.maximum(sel_t, minmask.astype(f32))

    w = jax.nn.softmax(fw_ref[...], axis=0)
    coef_t = jnp.where(sel_t > 0.5, sel_t, 0.0) * w

    mods = (m0, m1, m2, m3, m4, m5, m6, m7)
    terms = []
    for k in range(K):
        enc_t = jax.lax.dot_general(encw_ref[k].astype(jnp.bfloat16),
                                    mods[k][...].astype(jnp.bfloat16),
                                    dn_lane_lane,
                                    preferred_element_type=f32)
        terms.append(coef_t[k:k + 1, :] * enc_t)
    while len(terms) > 1:
        terms = [terms[i] + terms[i + 1] for i in range(0, len(terms), 2)]
    acc_t = terms[0]

    bias = jnp.dot(coef_t.T, encb_ref[...], preferred_element_type=f32)
    fused_ref[...] = acc_t.T + bias
    sel_ref[...] = sel_t.T
    probs_ref[...] = probs_t.T


@jax.jit
def kernel(context, mod_0, mod_1, mod_2, mod_3, mod_4, mod_5, mod_6, mod_7,
           r_w1, r_b1, ln_g, ln_b, r_w2, r_b2, r_w3, r_b3, prior, enc_W,
           enc_b, fusion_w):
    f32 = jnp.float32
    u = jax.random.uniform(jax.random.key(1234), (B, K), dtype=f32)
    g_t = (-jnp.log(-jnp.log(u + 1e-8) + 1e-8)).T

    row = lambda shape: pl.BlockSpec(shape, lambda i: (i, 0))
    colblk = lambda shape: pl.BlockSpec(shape, lambda i: (0, i))
    full2 = lambda shape: pl.BlockSpec(shape, lambda i: (0, 0))

    grid = B // ROWS
    out_shapes = (
        jax.ShapeDtypeStruct((B, H), f32),
        jax.ShapeDtypeStruct((B, K), f32),
        jax.ShapeDtypeStruct((B, K), f32),
    )
    in_specs = (
        [row((ROWS, CTX))] + [row((ROWS, D))] * K + [colblk((K, ROWS))] +
        [full2((RH, CTX)), full2((RH, 1)), full2((RH, 1)), full2((RH, 1)),
         full2((RH // 2, RH)), full2((RH // 2, 1)), full2((K, RH // 2)),
         full2((K, 1)), full2((K, 1)),
         pl.BlockSpec((K, H, D), lambda i: (0, 0, 0)), full2((K, H)),
         full2((K, 1))]
    )
    out_specs = (row((ROWS, H)), row((ROWS, K)), row((ROWS, K)))

    fused, sel, probs = pl.pallas_call(
        _fused_kernel,
        grid=(grid,),
        in_specs=in_specs,
        out_specs=out_specs,
        out_shape=out_shapes,
        compiler_params=pltpu.CompilerParams(
            dimension_semantics=("parallel",)),
    )(context, mod_0, mod_1, mod_2, mod_3, mod_4, mod_5, mod_6, mod_7, g_t,
      r_w1, r_b1.reshape(RH, 1), ln_g.reshape(RH, 1), ln_b.reshape(RH, 1),
      r_w2, r_b2.reshape(RH // 2, 1), r_w3, r_b3.reshape(K, 1),
      prior.reshape(K, 1), enc_W, enc_b, fusion_w.reshape(K, 1))
    return fused, sel, probs


# speedup vs baseline: 1.7684x; 1.0115x over previous
import jax
import jax.numpy as jnp
from jax.experimental import pallas as pl
from jax.experimental.pallas import tpu as pltpu

B = 16384
CTX = 128
D = 128
H = 128
K = 8
RH = 64
ROWS = 2048


def _fused_kernel(ctx_ref, m0, m1, m2, m3, m4, m5, m6, m7, gt_ref,
                  rw1_ref, rb1_ref, lng_ref, lnb_ref, rw2_ref, rb2_ref,
                  rw3_ref, rb3_ref, prior_ref, encw_ref, encb_ref, fw_ref,
                  fused_ref, sel_ref, probs_ref):
    f32 = jnp.float32
    dn_lane_lane = (((1,), (1,)), ((), ()))
    dn_lane_sub = (((1,), (0,)), ((), ()))

    ht = jax.lax.dot_general(rw1_ref[...], ctx_ref[...], dn_lane_lane,
                             preferred_element_type=f32) + rb1_ref[...]
    mu = jnp.mean(ht, axis=0, keepdims=True)
    var = jnp.mean((ht - mu) ** 2, axis=0, keepdims=True)
    ht = (ht - mu) / jnp.sqrt(var + 1e-5) * lng_ref[...] + lnb_ref[...]
    ht = jax.nn.relu(ht)
    h2t = jax.nn.relu(
        jax.lax.dot_general(rw2_ref[...], ht, dn_lane_sub,
                            preferred_element_type=f32) + rb2_ref[...])
    logits = jax.lax.dot_general(rw3_ref[...], h2t, dn_lane_sub,
                                 preferred_element_type=f32) + rb3_ref[...]
    logits = logits + prior_ref[...]
    probs_t = jax.nn.sigmoid(logits)
    sel_t = jax.nn.sigmoid(logits + gt_ref[...])

    iota = jax.lax.broadcasted_iota(jnp.int32, (K, ROWS), 0)
    m1v = jnp.max(probs_t, axis=0, keepdims=True)
    i1 = jnp.min(jnp.where(probs_t == m1v, iota, K), axis=0, keepdims=True)
    p2 = jnp.where(iota == i1, -jnp.inf, probs_t)
    m2v = jnp.max(p2, axis=0, keepdims=True)
    i2 = jnp.min(jnp.where(p2 == m2v, iota, K), axis=0, keepdims=True)
    minmask = (iota == i1) | (iota == i2)
    sel_t = jnp.maximum(sel_t, minmask.astype(f32))

    w = jax.nn.softmax(fw_ref[...], axis=0)
    coef_t = jnp.where(sel_t > 0.5, sel_t, 0.0) * w

    mods = (m0, m1, m2, m3, m4, m5, m6, m7)
    acc_t = None
    for k in range(K):
        enc_t = jax.lax.dot_general(encw_ref[k].astype(jnp.bfloat16),
                                    mods[k][...].astype(jnp.bfloat16),
                                    dn_lane_lane,
                                    preferred_element_type=f32)
        term = coef_t[k:k + 1, :] * enc_t
        acc_t = term if acc_t is None else acc_t + term

    bias = jnp.dot(coef_t.T, encb_ref[...], preferred_element_type=f32)
    fused_ref[...] = acc_t.T + bias
    sel_ref[...] = sel_t.T
    probs_ref[...] = probs_t.T


@jax.jit
def kernel(context, mod_0, mod_1, mod_2, mod_3, mod_4, mod_5, mod_6, mod_7,
           r_w1, r_b1, ln_g, ln_b, r_w2, r_b2, r_w3, r_b3, prior, enc_W,
           enc_b, fusion_w):
    f32 = jnp.float32
    u = jax.random.uniform(jax.random.key(1234), (B, K), dtype=f32)
    g_t = (-jnp.log(-jnp.log(u + 1e-8) + 1e-8)).T

    row = lambda shape: pl.BlockSpec(shape, lambda i: (i, 0))
    colblk = lambda shape: pl.BlockSpec(shape, lambda i: (0, i))
    full2 = lambda shape: pl.BlockSpec(shape, lambda i: (0, 0))

    grid = B // ROWS
    out_shapes = (
        jax.ShapeDtypeStruct((B, H), f32),
        jax.ShapeDtypeStruct((B, K), f32),
        jax.ShapeDtypeStruct((B, K), f32),
    )
    in_specs = (
        [row((ROWS, CTX))] + [row((ROWS, D))] * K + [colblk((K, ROWS))] +
        [full2((RH, CTX)), full2((RH, 1)), full2((RH, 1)), full2((RH, 1)),
         full2((RH // 2, RH)), full2((RH // 2, 1)), full2((K, RH // 2)),
         full2((K, 1)), full2((K, 1)),
         pl.BlockSpec((K, H, D), lambda i: (0, 0, 0)), full2((K, H)),
         full2((K, 1))]
    )
    out_specs = (row((ROWS, H)), row((ROWS, K)), row((ROWS, K)))

    fused, sel, probs = pl.pallas_call(
        _fused_kernel,
        grid=(grid,),
        in_specs=in_specs,
        out_specs=out_specs,
        out_shape=out_shapes,
        compiler_params=pltpu.CompilerParams(
            dimension_semantics=("parallel",)),
    )(context, mod_0, mod_1, mod_2, mod_3, mod_4, mod_5, mod_6, mod_7, g_t,
      r_w1, r_b1.reshape(RH, 1), ln_g.reshape(RH, 1), ln_b.reshape(RH, 1),
      r_w2, r_b2.reshape(RH // 2, 1), r_w3, r_b3.reshape(K, 1),
      prior.reshape(K, 1), enc_W, enc_b, fusion_w.reshape(K, 1))
    return fused, sel, probs
